# baseline (speedup 1.0000x reference)
import functools

import jax
import jax.numpy as jnp
from jax import lax
from jax.experimental import pallas as pl
from jax.experimental.pallas import tpu as pltpu
from jax.experimental.pallas import tpu_sc as plsc

B = 2048
DIM = 1024
E = 8
H = 16
DH = 64
EPAD = 128
TBLK = 256
NEG = -1e30

NPAIR = 2 * B
TM = 128
NT = NPAIR // TM + E - 1
NT = 40
CAP = NT * TM
NW = 32
PP = NPAIR // NW
TOKW = B // NW


GBLK = 256


def _gate_body(x_ref, wgt_ref, bg_ref, eb_ref, idx_ref, val_ref, cnt_ref):
    logits = jnp.dot(x_ref[...], wgt_ref[...],
                     preferred_element_type=jnp.float32) + bg_ref[0]
    m = jnp.max(logits, axis=-1, keepdims=True)
    p = jnp.exp(logits - m)
    probs = p / jnp.sum(p, axis=-1, keepdims=True)
    scores = probs + eb_ref[0]
    lane = jax.lax.broadcasted_iota(jnp.int32, (GBLK, EPAD), 1)
    v1 = jnp.max(scores, axis=-1, keepdims=True)
    i1 = jnp.min(jnp.where(scores == v1, lane, EPAD), axis=-1, keepdims=True)
    s2 = jnp.where(lane == i1, NEG, scores)
    v2 = jnp.max(s2, axis=-1, keepdims=True)
    i2 = jnp.min(jnp.where(s2 == v2, lane, EPAD), axis=-1, keepdims=True)
    idx_ref[...] = jnp.where(lane == 0, i1, jnp.where(lane == 1, i2, 0))
    val_ref[...] = jnp.where(lane == 0, v1, jnp.where(lane == 1, v2, 0.0))
    onehot = (lane == i1).astype(jnp.int32) + (lane == i2).astype(jnp.int32)
    for i in range(GBLK // TOKW):
        cnt_ref[i] = jnp.sum(onehot[i * TOKW:(i + 1) * TOKW], axis=0,
                             keepdims=True)


def _gating(x, Wg, bg, expert_bias):
    wgt = jnp.pad(Wg.T, ((0, 0), (0, EPAD - E)))
    bgp = jnp.pad(bg, (0, EPAD - E), constant_values=NEG)[None, :]
    ebp = jnp.pad(expert_bias, (0, EPAD - E), constant_values=NEG)[None, :]
    nw_blk = GBLK // TOKW
    idx2, val2, counts3 = pl.pallas_call(
        _gate_body,
        grid=(B // GBLK,),
        in_specs=[
            pl.BlockSpec((GBLK, DIM), lambda t: (t, 0)),
            pl.BlockSpec((DIM, EPAD), lambda t: (0, 0)),
            pl.BlockSpec((1, EPAD), lambda t: (0, 0)),
            pl.BlockSpec((1, EPAD), lambda t: (0, 0)),
        ],
        out_specs=[
            pl.BlockSpec((GBLK, EPAD), lambda t: (t, 0)),
            pl.BlockSpec((GBLK, EPAD), lambda t: (t, 0)),
            pl.BlockSpec((nw_blk, 1, EPAD), lambda t: (t, 0, 0)),
        ],
        out_shape=[
            jax.ShapeDtypeStruct((B, EPAD), jnp.int32),
            jax.ShapeDtypeStruct((B, EPAD), jnp.float32),
            jax.ShapeDtypeStruct((NW, 1, EPAD), jnp.int32),
        ],
    )(x, wgt, bgp, ebp)
    eidx = idx2[:, :2].reshape(NPAIR)
    tvals = val2[:, :2].reshape(NPAIR)
    return eidx, tvals, counts3


_MESH = plsc.VectorSubcoreMesh(core_axis_name="c", subcore_axis_name="s")


def _wid():
    return lax.axis_index("s") * 2 + lax.axis_index("c")


_GTR = lax.GatherDimensionNumbers(
    offset_dims=(), collapsed_slice_dims=(0,), start_index_map=(0,))


def _take16(x, idx):
    return lax.gather(x, idx[:, None], _GTR, slice_sizes=(1,),
                      mode=lax.GatherScatterMode.PROMISE_IN_BOUNDS)


def _psum16(x):
    idx = lax.iota(jnp.int32, 16)
    for k in (1, 2, 4, 8):
        sh = _take16(x, jnp.maximum(idx - k, 0))
        x = x + jnp.where(idx >= k, sh, 0)
    return x


def _b2_body(eidx_hbm, counts_hbm, x_hbm, y_hbm,
             xs_hbm, ys_hbm, pos_hbm, texp_hbm,
             ev_buf, acnt, pos_buf, tex_buf, tok4, pos4,
             rows0, rows1, sg0, sg1, ss0, ss1):
    w = _wid()
    base = w * PP
    pltpu.sync_copy(eidx_hbm.at[pl.ds(base, PP)], ev_buf)
    pltpu.sync_copy(counts_hbm, acnt)

    tot = jnp.zeros((16,), jnp.int32)
    before = jnp.zeros((16,), jnp.int32)
    for ww in range(NW):
        row = acnt[ww, 0, pl.ds(0, 16)]
        sel = (jnp.asarray(ww, jnp.int32) < w).astype(jnp.int32)
        before = before + sel * row
        tot = tot + row
    ptile = lax.shift_right_logical(tot + (TM - 1), 7)
    inc = _psum16(ptile)
    tstart = inc - ptile
    mb = tstart * TM + before

    mbs = [mb[e] for e in range(E)]
    rc = [jnp.zeros((), jnp.int32) for _ in range(E)]
    for c in range(PP // 16):
        ev = ev_buf[pl.ds(c * 16, 16)]
        posv = jnp.zeros((16,), jnp.int32)
        for e in range(E):
            msk = ev == e
            pre = _psum16(jnp.where(msk, 1, 0))
            slot = (mbs[e] + rc[e] - 1) + pre
            posv = jnp.where(msk, slot, posv)
            rc[e] = rc[e] + pre[15]
        pos_buf[pl.ds(c * 16, 16)] = posv
    pltpu.sync_copy(pos_buf, pos_hbm.at[pl.ds(base, PP)])

    @pl.when(w == 0)
    def _():
        sts = [tstart[e] for e in range(E)]
        for j in range(3):
            r_io = lax.iota(jnp.int32, 16) + j * 16
            ex = jnp.zeros((16,), jnp.int32)
            for e in range(1, E):
                ex = ex + jnp.where(r_io >= sts[e], 1, 0)
            tex_buf[pl.ds(j * 16, 16)] = ex
        pltpu.sync_copy(tex_buf, texp_hbm)

    for c in range(PP // 16):
        tokv = lax.shift_right_logical(
            lax.iota(jnp.int32, 16) + (base + c * 16), 1)
        tok4[c // 2, pl.ds((c % 2) * 16, 16)] = tokv
        pos4[c // 2, pl.ds((c % 2) * 16, 16)] = pos_buf[pl.ds(c * 16, 16)]

    jobs = [(x_hbm, xs_hbm), (y_hbm, ys_hbm)]
    seq = [(a, r) for r in range(4) for a in range(2)]
    bufs = [rows0, rows1]
    gsem = [sg0, sg1]
    ssem = [ss0, ss1]
    pending_s = [None, None]

    def start_gather(j):
        a, r = seq[j]
        b = j % 2
        if pending_s[b] is not None:
            pending_s[b].wait()
        return pltpu.async_copy(jobs[a][0].at[tok4.at[r]], bufs[b], gsem[b])

    gh = [start_gather(0), start_gather(1)]
    for j in range(8):
        a, r = seq[j]
        b = j % 2
        gh[b].wait()
        pending_s[b] = pltpu.async_copy(bufs[b], jobs[a][1].at[pos4.at[r]],
                                        ssem[b])
        if j + 2 < 8:
            gh[b] = start_gather(j + 2)
    pending_s[0].wait()
    pending_s[1].wait()


def _b2(eidx, counts, x, y):
    f = pl.kernel(
        _b2_body,
        out_type=[
            jax.ShapeDtypeStruct((CAP, DIM), jnp.float32),
            jax.ShapeDtypeStruct((CAP, DIM), jnp.float32),
            jax.ShapeDtypeStruct((NPAIR,), jnp.int32),
            jax.ShapeDtypeStruct((48,), jnp.int32),
        ],
        mesh=_MESH,
        scratch_types=[
            pltpu.VMEM((PP,), jnp.int32),
            pltpu.VMEM((NW, 1, EPAD), jnp.int32),
            pltpu.VMEM((PP,), jnp.int32),
            pltpu.VMEM((48,), jnp.int32),
            pltpu.VMEM((4, 32), jnp.int32),
            pltpu.VMEM((4, 32), jnp.int32),
            pltpu.VMEM((32, DIM), jnp.float32),
            pltpu.VMEM((32, DIM), jnp.float32),
            pltpu.SemaphoreType.DMA,
            pltpu.SemaphoreType.DMA,
            pltpu.SemaphoreType.DMA,
            pltpu.SemaphoreType.DMA,
        ],
    )
    return f(eidx, counts, x, y)


def _c_body(texp_ref, ys_ref, xs_ref, w_ref, out_ref):
    w3 = w_ref[0]
    out_ref[:, :DIM] = jnp.dot(ys_ref[...], w3[:, :DIM],
                               preferred_element_type=jnp.float32)
    out_ref[:, DIM:] = jnp.dot(xs_ref[...], w3[:, DIM:],
                               preferred_element_type=jnp.float32)


def _c(texp, ys_s, xs_s, W_qkv):
    grid_spec = pltpu.PrefetchScalarGridSpec(
        num_scalar_prefetch=1,
        grid=(NT,),
        in_specs=[
            pl.BlockSpec((TM, DIM), lambda r, texp: (r, 0)),
            pl.BlockSpec((TM, DIM), lambda r, texp: (r, 0)),
            pl.BlockSpec((1, DIM, 3 * DIM), lambda r, texp: (texp[r], 0, 0)),
        ],
        out_specs=pl.BlockSpec((TM, 3 * DIM), lambda r, texp: (r, 0)),
    )
    return pl.pallas_call(
        _c_body,
        grid_spec=grid_spec,
        out_shape=jax.ShapeDtypeStruct((CAP, 3 * DIM), jnp.float32),
        compiler_params=pltpu.CompilerParams(
            dimension_semantics=("arbitrary",)),
    )(texp, ys_s, xs_s, W_qkv)


HALF = 2
HB = B // HALF
PPH = 2 * HB // NW
TOKWH = HB // NW


def _d_body(pairout_hbm, pos_hbm, tv_hbm, qr_hbm, kr_hbm, vr_hbm,
            pos_buf, tv_buf, rows0, rows1, obq, obk, obv, sem0, sem1):
    w = _wid()
    pltpu.sync_copy(pos_hbm.at[pl.ds(w * PPH, PPH)], pos_buf)
    pltpu.sync_copy(tv_hbm.at[pl.ds(w * PPH, PPH)], tv_buf)
    obs = [obq, obk, obv]
    outs = [qr_hbm, kr_hbm, vr_hbm]
    bufs = [rows0, rows1]
    sems = [sem0, sem1]

    def start(c):
        posv = pos_buf[pl.ds(c * 16, 16)]
        return pltpu.async_copy(pairout_hbm.at[posv], bufs[c % 2],
                                sems[c % 2])

    nch = TOKWH // 8
    gh = [start(0), start(1)]
    for c in range(nch):
        rows = bufs[c % 2]
        gh[c % 2].wait()
        gv = tv_buf[pl.ds(c * 16, 16)]
        g = [gv[j] for j in range(16)]

        for pi in range(3):
            def body(s, _, rows=rows, g=g, pi=pi):
                for t in range(8):
                    a = rows[2 * t, pl.ds(pi * DIM + s * 16, 16)]
                    b = rows[2 * t + 1, pl.ds(pi * DIM + s * 16, 16)]
                    obs[pi][t, pl.ds(s * 16, 16)] = (
                        g[2 * t] * a + g[2 * t + 1] * b)
                return 0

            lax.fori_loop(0, DIM // 16, body, 0)
        rb = w * TOKWH + c * 8
        for pi in range(3):
            pltpu.sync_copy(obs[pi], outs[pi].at[pl.ds(rb, 8)])
        if c + 2 < nch:
            gh[c % 2] = start(c + 2)


def _d(pairout, pos, tvals):
    f = pl.kernel(
        _d_body,
        out_type=[
            jax.ShapeDtypeStruct((HB, DIM), jnp.float32),
            jax.ShapeDtypeStruct((HB, DIM), jnp.float32),
            jax.ShapeDtypeStruct((HB, DIM), jnp.float32),
        ],
        mesh=_MESH,
        scratch_types=[
            pltpu.VMEM((PPH,), jnp.int32),
            pltpu.VMEM((PPH,), jnp.float32),
            pltpu.VMEM((16, 3 * DIM), jnp.float32),
            pltpu.VMEM((16, 3 * DIM), jnp.float32),
            pltpu.VMEM((8, DIM), jnp.float32),
            pltpu.VMEM((8, DIM), jnp.float32),
            pltpu.VMEM((8, DIM), jnp.float32),
            pltpu.SemaphoreType.DMA,
            pltpu.SemaphoreType.DMA,
        ],
    )
    return f(pairout, pos, tvals)


def _attn_body(q_ref, k_ref, v_ref, o_ref):
    scale = DH ** -0.5
    r = jax.lax.broadcasted_iota(jnp.int32, (1, 128, 128), 1) // H
    c = jax.lax.broadcasted_iota(jnp.int32, (1, 128, 128), 2) // H
    amask = jnp.where(r == c, 0.0, NEG).astype(jnp.float32)
    nb = q_ref.shape[0] // 128
    q3 = q_ref[...].reshape(nb, 128, DH)
    k3 = k_ref[...].reshape(nb, 128, DH)
    v3 = v_ref[...].reshape(nb, 128, DH)
    s = jax.lax.dot_general(q3, k3, (((2,), (2,)), ((0,), (0,))),
                            preferred_element_type=jnp.float32)
    s = s * scale + amask
    m = jnp.max(s, axis=-1, keepdims=True)
    p = jnp.exp(s - m)
    attn = p / jnp.sum(p, axis=-1, keepdims=True)
    o = jax.lax.dot_general(attn, v3, (((2,), (1,)), ((0,), (0,))),
                            preferred_element_type=jnp.float32)
    o_ref[...] = o.reshape(nb * 128, DH)


def _proj_body(o_ref, wp_ref, bp_ref, out_ref):
    out_ref[...] = jnp.dot(o_ref[...], wp_ref[...],
                           preferred_element_type=jnp.float32) + bp_ref[0]


def _attention_and_proj(qp, kp, vp, wp_perm, bp):
    q_r = qp.reshape(HB * H, DH)
    k_r = kp.reshape(HB * H, DH)
    v_r = vp.reshape(HB * H, DH)
    RB = 1024
    o_r = pl.pallas_call(
        _attn_body,
        grid=(HB * H // RB,),
        in_specs=[
            pl.BlockSpec((RB, DH), lambda i: (i, 0)),
            pl.BlockSpec((RB, DH), lambda i: (i, 0)),
            pl.BlockSpec((RB, DH), lambda i: (i, 0)),
        ],
        out_specs=pl.BlockSpec((RB, DH), lambda i: (i, 0)),
        out_shape=jax.ShapeDtypeStruct((HB * H, DH), jnp.float32),
    )(q_r, k_r, v_r)

    o_flat = o_r.reshape(HB, DIM)
    return pl.pallas_call(
        _proj_body,
        grid=(HB // TBLK,),
        in_specs=[
            pl.BlockSpec((TBLK, DIM), lambda i: (i, 0)),
            pl.BlockSpec((DIM, DIM), lambda i: (0, 0)),
            pl.BlockSpec((1, DIM), lambda i: (0, 0)),
        ],
        out_specs=pl.BlockSpec((TBLK, DIM), lambda i: (i, 0)),
        out_shape=jax.ShapeDtypeStruct((HB, DIM), jnp.float32),
    )(o_flat, wp_perm, bp[None, :])


def kernel(x, y, W_qkv, Wg, bg, Wp, bp, expert_bias):
    eidx, tvals, counts = _gating(x, Wg, bg, expert_bias)
    xs_s, ys_s, pos, texp = _b2(eidx, counts, x, y)
    pairout = _c(texp, ys_s, xs_s, W_qkv)
    wp_perm = Wp.T.reshape(DH, H, DIM).transpose(1, 0, 2).reshape(DIM, DIM)
    outs = []
    for h in range(HALF):
        pos_h = pos[h * 2 * HB:(h + 1) * 2 * HB]
        tv_h = tvals[h * 2 * HB:(h + 1) * 2 * HB]
        qp, kp, vp = _d(pairout, pos_h, tv_h)
        outs.append(_attention_and_proj(qp, kp, vp, wp_perm, bp))
    return jnp.concatenate(outs, axis=0)


# speedup vs baseline: 1.0596x; 1.0596x over previous
import functools

import jax
import jax.numpy as jnp
from jax import lax
from jax.experimental import pallas as pl
from jax.experimental.pallas import tpu as pltpu
from jax.experimental.pallas import tpu_sc as plsc

B = 2048
DIM = 1024
E = 8
H = 16
DH = 64
EPAD = 128
TBLK = 256
NEG = -1e30

NPAIR = 2 * B
TM = 128
NT = NPAIR // TM + E - 1
NT = 40
CAP = NT * TM
NW = 32
PP = NPAIR // NW
TOKW = B // NW


GBLK = 256


def _gate_body(x_ref, wgt_ref, bg_ref, eb_ref, idx_ref, val_ref, cnt_ref):
    logits = jnp.dot(x_ref[...], wgt_ref[...],
                     preferred_element_type=jnp.float32) + bg_ref[0]
    m = jnp.max(logits, axis=-1, keepdims=True)
    p = jnp.exp(logits - m)
    probs = p / jnp.sum(p, axis=-1, keepdims=True)
    scores = probs + eb_ref[0]
    lane = jax.lax.broadcasted_iota(jnp.int32, (GBLK, EPAD), 1)
    v1 = jnp.max(scores, axis=-1, keepdims=True)
    i1 = jnp.min(jnp.where(scores == v1, lane, EPAD), axis=-1, keepdims=True)
    s2 = jnp.where(lane == i1, NEG, scores)
    v2 = jnp.max(s2, axis=-1, keepdims=True)
    i2 = jnp.min(jnp.where(s2 == v2, lane, EPAD), axis=-1, keepdims=True)
    idx_ref[...] = jnp.where(lane == 0, i1, jnp.where(lane == 1, i2, 0))
    val_ref[...] = jnp.where(lane == 0, v1, jnp.where(lane == 1, v2, 0.0))
    onehot = (lane == i1).astype(jnp.int32) + (lane == i2).astype(jnp.int32)
    for i in range(GBLK // TOKW):
        cnt_ref[i] = jnp.sum(onehot[i * TOKW:(i + 1) * TOKW], axis=0,
                             keepdims=True)


def _gating(x, Wg, bg, expert_bias):
    wgt = jnp.pad(Wg.T, ((0, 0), (0, EPAD - E)))
    bgp = jnp.pad(bg, (0, EPAD - E), constant_values=NEG)[None, :]
    ebp = jnp.pad(expert_bias, (0, EPAD - E), constant_values=NEG)[None, :]
    nw_blk = GBLK // TOKW
    idx2, val2, counts3 = pl.pallas_call(
        _gate_body,
        grid=(B // GBLK,),
        in_specs=[
            pl.BlockSpec((GBLK, DIM), lambda t: (t, 0)),
            pl.BlockSpec((DIM, EPAD), lambda t: (0, 0)),
            pl.BlockSpec((1, EPAD), lambda t: (0, 0)),
            pl.BlockSpec((1, EPAD), lambda t: (0, 0)),
        ],
        out_specs=[
            pl.BlockSpec((GBLK, EPAD), lambda t: (t, 0)),
            pl.BlockSpec((GBLK, EPAD), lambda t: (t, 0)),
            pl.BlockSpec((nw_blk, 1, EPAD), lambda t: (t, 0, 0)),
        ],
        out_shape=[
            jax.ShapeDtypeStruct((B, EPAD), jnp.int32),
            jax.ShapeDtypeStruct((B, EPAD), jnp.float32),
            jax.ShapeDtypeStruct((NW, 1, EPAD), jnp.int32),
        ],
    )(x, wgt, bgp, ebp)
    eidx = idx2[:, :2].reshape(NPAIR)
    tvals = val2[:, :2].reshape(NPAIR)
    return eidx, tvals, counts3


_MESH = plsc.VectorSubcoreMesh(core_axis_name="c", subcore_axis_name="s")


def _wid():
    return lax.axis_index("s") * 2 + lax.axis_index("c")


_GTR = lax.GatherDimensionNumbers(
    offset_dims=(), collapsed_slice_dims=(0,), start_index_map=(0,))


def _take16(x, idx):
    return lax.gather(x, idx[:, None], _GTR, slice_sizes=(1,),
                      mode=lax.GatherScatterMode.PROMISE_IN_BOUNDS)


def _psum16(x):
    idx = lax.iota(jnp.int32, 16)
    for k in (1, 2, 4, 8):
        sh = _take16(x, jnp.maximum(idx - k, 0))
        x = x + jnp.where(idx >= k, sh, 0)
    return x


def _b2_body(eidx_hbm, counts_hbm, x_hbm, y_hbm,
             xs_hbm, ys_hbm, pos_hbm, texp_hbm,
             ev_buf, acnt, pos_buf, tex_buf, tok4, pos4,
             rows0, rows1, sg0, sg1, ss0, ss1):
    w = _wid()
    base = w * PP
    pltpu.sync_copy(eidx_hbm.at[pl.ds(base, PP)], ev_buf)
    pltpu.sync_copy(counts_hbm, acnt)

    tot = jnp.zeros((16,), jnp.int32)
    before = jnp.zeros((16,), jnp.int32)
    for ww in range(NW):
        row = acnt[ww, 0, pl.ds(0, 16)]
        sel = (jnp.asarray(ww, jnp.int32) < w).astype(jnp.int32)
        before = before + sel * row
        tot = tot + row
    ptile = lax.shift_right_logical(tot + (TM - 1), 7)
    inc = _psum16(ptile)
    tstart = inc - ptile
    mb = tstart * TM + before

    mbs = [mb[e] for e in range(E)]
    rc = [jnp.zeros((), jnp.int32) for _ in range(E)]
    for c in range(PP // 16):
        ev = ev_buf[pl.ds(c * 16, 16)]
        posv = jnp.zeros((16,), jnp.int32)
        for e in range(E):
            msk = ev == e
            pre = _psum16(jnp.where(msk, 1, 0))
            slot = (mbs[e] + rc[e] - 1) + pre
            posv = jnp.where(msk, slot, posv)
            rc[e] = rc[e] + pre[15]
        pos_buf[pl.ds(c * 16, 16)] = posv
    pltpu.sync_copy(pos_buf, pos_hbm.at[pl.ds(base, PP)])

    @pl.when(w == 0)
    def _():
        sts = [tstart[e] for e in range(E)]
        for j in range(3):
            r_io = lax.iota(jnp.int32, 16) + j * 16
            ex = jnp.zeros((16,), jnp.int32)
            for e in range(1, E):
                ex = ex + jnp.where(r_io >= sts[e], 1, 0)
            tex_buf[pl.ds(j * 16, 16)] = ex
        pltpu.sync_copy(tex_buf, texp_hbm)

    for c in range(PP // 16):
        tokv = lax.shift_right_logical(
            lax.iota(jnp.int32, 16) + (base + c * 16), 1)
        tok4[c // 2, pl.ds((c % 2) * 16, 16)] = tokv
        pos4[c // 2, pl.ds((c % 2) * 16, 16)] = pos_buf[pl.ds(c * 16, 16)]

    jobs = [(x_hbm, xs_hbm), (y_hbm, ys_hbm)]
    seq = [(a, r) for r in range(4) for a in range(2)]
    bufs = [rows0, rows1]
    gsem = [sg0, sg1]
    ssem = [ss0, ss1]
    pending_s = [None, None]

    def start_gather(j):
        a, r = seq[j]
        b = j % 2
        if pending_s[b] is not None:
            pending_s[b].wait()
        return pltpu.async_copy(jobs[a][0].at[tok4.at[r]], bufs[b], gsem[b])

    gh = [start_gather(0), start_gather(1)]
    for j in range(8):
        a, r = seq[j]
        b = j % 2
        gh[b].wait()
        pending_s[b] = pltpu.async_copy(bufs[b], jobs[a][1].at[pos4.at[r]],
                                        ssem[b])
        if j + 2 < 8:
            gh[b] = start_gather(j + 2)
    pending_s[0].wait()
    pending_s[1].wait()


def _b2(eidx, counts, x, y):
    f = pl.kernel(
        _b2_body,
        out_type=[
            jax.ShapeDtypeStruct((CAP, DIM), jnp.float32),
            jax.ShapeDtypeStruct((CAP, DIM), jnp.float32),
            jax.ShapeDtypeStruct((NPAIR,), jnp.int32),
            jax.ShapeDtypeStruct((48,), jnp.int32),
        ],
        mesh=_MESH,
        scratch_types=[
            pltpu.VMEM((PP,), jnp.int32),
            pltpu.VMEM((NW, 1, EPAD), jnp.int32),
            pltpu.VMEM((PP,), jnp.int32),
            pltpu.VMEM((48,), jnp.int32),
            pltpu.VMEM((4, 32), jnp.int32),
            pltpu.VMEM((4, 32), jnp.int32),
            pltpu.VMEM((32, DIM), jnp.float32),
            pltpu.VMEM((32, DIM), jnp.float32),
            pltpu.SemaphoreType.DMA,
            pltpu.SemaphoreType.DMA,
            pltpu.SemaphoreType.DMA,
            pltpu.SemaphoreType.DMA,
        ],
    )
    return f(eidx, counts, x, y)


def _c_body(texp_ref, ys_ref, xs_ref, w_ref, out_ref):
    w3 = w_ref[0]
    out_ref[:, :DIM] = jnp.dot(ys_ref[...], w3[:, :DIM],
                               preferred_element_type=jnp.float32)
    out_ref[:, DIM:] = jnp.dot(xs_ref[...], w3[:, DIM:],
                               preferred_element_type=jnp.float32)


def _c(texp, ys_s, xs_s, W_qkv):
    grid_spec = pltpu.PrefetchScalarGridSpec(
        num_scalar_prefetch=1,
        grid=(NT,),
        in_specs=[
            pl.BlockSpec((TM, DIM), lambda r, texp: (r, 0)),
            pl.BlockSpec((TM, DIM), lambda r, texp: (r, 0)),
            pl.BlockSpec((1, DIM, 3 * DIM), lambda r, texp: (texp[r], 0, 0)),
        ],
        out_specs=pl.BlockSpec((TM, 3 * DIM), lambda r, texp: (r, 0)),
    )
    return pl.pallas_call(
        _c_body,
        grid_spec=grid_spec,
        out_shape=jax.ShapeDtypeStruct((CAP, 3 * DIM), jnp.float32),
        compiler_params=pltpu.CompilerParams(
            dimension_semantics=("arbitrary",)),
    )(texp, ys_s, xs_s, W_qkv)


def _d_body(pairout_hbm, pos_hbm, tv_hbm, qr_hbm, kr_hbm, vr_hbm,
            pos_buf, tv_buf, rows0, rows1, obq, obk, obv, sem0, sem1):
    w = _wid()
    pltpu.sync_copy(pos_hbm.at[pl.ds(w * PP, PP)], pos_buf)
    pltpu.sync_copy(tv_hbm.at[pl.ds(w * PP, PP)], tv_buf)
    obs = [obq, obk, obv]
    outs = [qr_hbm, kr_hbm, vr_hbm]
    bufs = [rows0, rows1]
    sems = [sem0, sem1]

    def start(c):
        posv = pos_buf[pl.ds(c * 16, 16)]
        return pltpu.async_copy(pairout_hbm.at[posv], bufs[c % 2],
                                sems[c % 2])

    nch = TOKW // 8
    gh = [start(0), start(1)]
    for c in range(nch):
        rows = bufs[c % 2]
        gh[c % 2].wait()
        gv = tv_buf[pl.ds(c * 16, 16)]
        g = [gv[j] for j in range(16)]

        for pi in range(3):
            def body(s, _, rows=rows, g=g, pi=pi):
                for t in range(8):
                    a = rows[2 * t, pl.ds(pi * DIM + s * 16, 16)]
                    b = rows[2 * t + 1, pl.ds(pi * DIM + s * 16, 16)]
                    obs[pi][t, pl.ds(s * 16, 16)] = (
                        g[2 * t] * a + g[2 * t + 1] * b)
                return 0

            lax.fori_loop(0, DIM // 16, body, 0)
        rb = w * TOKW + c * 8
        for pi in range(3):
            pltpu.sync_copy(obs[pi], outs[pi].at[pl.ds(rb, 8)])
        if c + 2 < nch:
            gh[c % 2] = start(c + 2)


def _d(pairout, pos, tvals):
    f = pl.kernel(
        _d_body,
        out_type=[
            jax.ShapeDtypeStruct((B, DIM), jnp.float32),
            jax.ShapeDtypeStruct((B, DIM), jnp.float32),
            jax.ShapeDtypeStruct((B, DIM), jnp.float32),
        ],
        mesh=_MESH,
        scratch_types=[
            pltpu.VMEM((PP,), jnp.int32),
            pltpu.VMEM((PP,), jnp.float32),
            pltpu.VMEM((16, 3 * DIM), jnp.float32),
            pltpu.VMEM((16, 3 * DIM), jnp.float32),
            pltpu.VMEM((8, DIM), jnp.float32),
            pltpu.VMEM((8, DIM), jnp.float32),
            pltpu.VMEM((8, DIM), jnp.float32),
            pltpu.SemaphoreType.DMA,
            pltpu.SemaphoreType.DMA,
        ],
    )
    return f(pairout, pos, tvals)


def _attn_body(q_ref, k_ref, v_ref, o_ref):
    scale = DH ** -0.5
    r = jax.lax.broadcasted_iota(jnp.int32, (1, 128, 128), 1) // H
    c = jax.lax.broadcasted_iota(jnp.int32, (1, 128, 128), 2) // H
    amask = jnp.where(r == c, 0.0, NEG).astype(jnp.float32)
    nb = q_ref.shape[0] // 128
    q3 = q_ref[...].reshape(nb, 128, DH)
    k3 = k_ref[...].reshape(nb, 128, DH)
    v3 = v_ref[...].reshape(nb, 128, DH)
    s = jax.lax.dot_general(q3, k3, (((2,), (2,)), ((0,), (0,))),
                            preferred_element_type=jnp.float32)
    s = s * scale + amask
    m = jnp.max(s, axis=-1, keepdims=True)
    p = jnp.exp(s - m)
    attn = p / jnp.sum(p, axis=-1, keepdims=True)
    o = jax.lax.dot_general(attn, v3, (((2,), (1,)), ((0,), (0,))),
                            preferred_element_type=jnp.float32)
    o_ref[...] = o.reshape(nb * 128, DH)


def _proj_body(o_ref, wp_ref, bp_ref, out_ref):
    out_ref[...] = jnp.dot(o_ref[...], wp_ref[...],
                           preferred_element_type=jnp.float32) + bp_ref[0]


def _attention_and_proj(qp, kp, vp, Wp, bp):
    q_r = qp.reshape(B * H, DH)
    k_r = kp.reshape(B * H, DH)
    v_r = vp.reshape(B * H, DH)
    RB = 4096
    o_r = pl.pallas_call(
        _attn_body,
        grid=(B * H // RB,),
        in_specs=[
            pl.BlockSpec((RB, DH), lambda i: (i, 0)),
            pl.BlockSpec((RB, DH), lambda i: (i, 0)),
            pl.BlockSpec((RB, DH), lambda i: (i, 0)),
        ],
        out_specs=pl.BlockSpec((RB, DH), lambda i: (i, 0)),
        out_shape=jax.ShapeDtypeStruct((B * H, DH), jnp.float32),
    )(q_r, k_r, v_r)

    o_flat = o_r.reshape(B, DIM)
    wp_perm = Wp.T.reshape(DH, H, DIM).transpose(1, 0, 2).reshape(DIM, DIM)
    return pl.pallas_call(
        _proj_body,
        grid=(B // TBLK,),
        in_specs=[
            pl.BlockSpec((TBLK, DIM), lambda i: (i, 0)),
            pl.BlockSpec((DIM, DIM), lambda i: (0, 0)),
            pl.BlockSpec((1, DIM), lambda i: (0, 0)),
        ],
        out_specs=pl.BlockSpec((TBLK, DIM), lambda i: (i, 0)),
        out_shape=jax.ShapeDtypeStruct((B, DIM), jnp.float32),
    )(o_flat, wp_perm, bp[None, :])


def kernel(x, y, W_qkv, Wg, bg, Wp, bp, expert_bias):
    eidx, tvals, counts = _gating(x, Wg, bg, expert_bias)
    xs_s, ys_s, pos, texp = _b2(eidx, counts, x, y)
    pairout = _c(texp, ys_s, xs_s, W_qkv)
    q_r, k_r, v_r = _d(pairout, pos, tvals)
    return _attention_and_proj(q_r, k_r, v_r, Wp, bp)


# speedup vs baseline: 1.0668x; 1.0068x over previous
import functools

import jax
import jax.numpy as jnp
from jax import lax
from jax.experimental import pallas as pl
from jax.experimental.pallas import tpu as pltpu
from jax.experimental.pallas import tpu_sc as plsc

B = 2048
DIM = 1024
E = 8
H = 16
DH = 64
EPAD = 128
TBLK = 256
NEG = -1e30

NPAIR = 2 * B
TM = 128
NT = NPAIR // TM + E - 1
NT = 40
CAP = NT * TM
NW = 32
PP = NPAIR // NW
TOKW = B // NW


GBLK = 256


def _gate_body(x_ref, wgt_ref, bg_ref, eb_ref, idx_ref, val_ref, cnt_ref):
    logits = jnp.dot(x_ref[...], wgt_ref[...],
                     preferred_element_type=jnp.float32) + bg_ref[0]
    m = jnp.max(logits, axis=-1, keepdims=True)
    p = jnp.exp(logits - m)
    probs = p / jnp.sum(p, axis=-1, keepdims=True)
    scores = probs + eb_ref[0]
    lane = jax.lax.broadcasted_iota(jnp.int32, (GBLK, EPAD), 1)
    v1 = jnp.max(scores, axis=-1, keepdims=True)
    i1 = jnp.min(jnp.where(scores == v1, lane, EPAD), axis=-1, keepdims=True)
    s2 = jnp.where(lane == i1, NEG, scores)
    v2 = jnp.max(s2, axis=-1, keepdims=True)
    i2 = jnp.min(jnp.where(s2 == v2, lane, EPAD), axis=-1, keepdims=True)
    idx_ref[...] = jnp.where(lane == 0, i1, jnp.where(lane == 1, i2, 0))
    val_ref[...] = jnp.where(lane == 0, v1, jnp.where(lane == 1, v2, 0.0))
    onehot = (lane == i1).astype(jnp.int32) + (lane == i2).astype(jnp.int32)
    for i in range(GBLK // TOKW):
        cnt_ref[i] = jnp.sum(onehot[i * TOKW:(i + 1) * TOKW], axis=0,
                             keepdims=True)


def _gating(x, Wg, bg, expert_bias):
    wgt = jnp.pad(Wg.T, ((0, 0), (0, EPAD - E)))
    bgp = jnp.pad(bg, (0, EPAD - E), constant_values=NEG)[None, :]
    ebp = jnp.pad(expert_bias, (0, EPAD - E), constant_values=NEG)[None, :]
    nw_blk = GBLK // TOKW
    idx2, val2, counts3 = pl.pallas_call(
        _gate_body,
        grid=(B // GBLK,),
        in_specs=[
            pl.BlockSpec((GBLK, DIM), lambda t: (t, 0)),
            pl.BlockSpec((DIM, EPAD), lambda t: (0, 0)),
            pl.BlockSpec((1, EPAD), lambda t: (0, 0)),
            pl.BlockSpec((1, EPAD), lambda t: (0, 0)),
        ],
        out_specs=[
            pl.BlockSpec((GBLK, EPAD), lambda t: (t, 0)),
            pl.BlockSpec((GBLK, EPAD), lambda t: (t, 0)),
            pl.BlockSpec((nw_blk, 1, EPAD), lambda t: (t, 0, 0)),
        ],
        out_shape=[
            jax.ShapeDtypeStruct((B, EPAD), jnp.int32),
            jax.ShapeDtypeStruct((B, EPAD), jnp.float32),
            jax.ShapeDtypeStruct((NW, 1, EPAD), jnp.int32),
        ],
    )(x, wgt, bgp, ebp)
    eidx = idx2[:, :2].reshape(NPAIR)
    tvals = val2[:, :2].reshape(NPAIR)
    return eidx, tvals, counts3


_MESH = plsc.VectorSubcoreMesh(core_axis_name="c", subcore_axis_name="s")


def _wid():
    return lax.axis_index("s") * 2 + lax.axis_index("c")


_GTR = lax.GatherDimensionNumbers(
    offset_dims=(), collapsed_slice_dims=(0,), start_index_map=(0,))


def _take16(x, idx):
    return lax.gather(x, idx[:, None], _GTR, slice_sizes=(1,),
                      mode=lax.GatherScatterMode.PROMISE_IN_BOUNDS)


def _psum16(x):
    idx = lax.iota(jnp.int32, 16)
    for k in (1, 2, 4, 8):
        sh = _take16(x, jnp.maximum(idx - k, 0))
        x = x + jnp.where(idx >= k, sh, 0)
    return x


def _b2_body(eidx_hbm, counts_hbm, x_hbm, y_hbm,
             xs_hbm, ys_hbm, pos_hbm, texp_hbm,
             ev_buf, acnt, pos_buf, tex_buf, tok4, pos4,
             rows0, rows1, sg0, sg1, ss0, ss1):
    w = _wid()
    base = w * PP
    pltpu.sync_copy(eidx_hbm.at[pl.ds(base, PP)], ev_buf)
    pltpu.sync_copy(counts_hbm, acnt)

    tot = jnp.zeros((16,), jnp.int32)
    before = jnp.zeros((16,), jnp.int32)
    for ww in range(NW):
        row = acnt[ww, 0, pl.ds(0, 16)]
        sel = (jnp.asarray(ww, jnp.int32) < w).astype(jnp.int32)
        before = before + sel * row
        tot = tot + row
    ptile = lax.shift_right_logical(tot + (TM - 1), 7)
    inc = _psum16(ptile)
    tstart = inc - ptile
    mb = tstart * TM + before

    mbs = [mb[e] for e in range(E)]
    rc = [jnp.zeros((), jnp.int32) for _ in range(E)]
    for c in range(PP // 16):
        ev = ev_buf[pl.ds(c * 16, 16)]
        posv = jnp.zeros((16,), jnp.int32)
        for e in range(E):
            msk = ev == e
            pre = _psum16(jnp.where(msk, 1, 0))
            slot = (mbs[e] + rc[e] - 1) + pre
            posv = jnp.where(msk, slot, posv)
            rc[e] = rc[e] + pre[15]
        pos_buf[pl.ds(c * 16, 16)] = posv
    pltpu.sync_copy(pos_buf, pos_hbm.at[pl.ds(base, PP)])

    @pl.when(w == 0)
    def _():
        sts = [tstart[e] for e in range(E)]
        for j in range(3):
            r_io = lax.iota(jnp.int32, 16) + j * 16
            ex = jnp.zeros((16,), jnp.int32)
            for e in range(1, E):
                ex = ex + jnp.where(r_io >= sts[e], 1, 0)
            tex_buf[pl.ds(j * 16, 16)] = ex
        pltpu.sync_copy(tex_buf, texp_hbm)

    for c in range(PP // 16):
        tokv = lax.shift_right_logical(
            lax.iota(jnp.int32, 16) + (base + c * 16), 1)
        tok4[c // 2, pl.ds((c % 2) * 16, 16)] = tokv
        pos4[c // 2, pl.ds((c % 2) * 16, 16)] = pos_buf[pl.ds(c * 16, 16)]

    jobs = [(x_hbm, xs_hbm), (y_hbm, ys_hbm)]
    seq = [(a, r) for r in range(4) for a in range(2)]
    bufs = [rows0, rows1]
    gsem = [sg0, sg1]
    ssem = [ss0, ss1]
    pending_s = [None, None]

    def start_gather(j):
        a, r = seq[j]
        b = j % 2
        if pending_s[b] is not None:
            pending_s[b].wait()
        return pltpu.async_copy(jobs[a][0].at[tok4.at[r]], bufs[b], gsem[b])

    gh = [start_gather(0), start_gather(1)]
    for j in range(8):
        a, r = seq[j]
        b = j % 2
        gh[b].wait()
        pending_s[b] = pltpu.async_copy(bufs[b], jobs[a][1].at[pos4.at[r]],
                                        ssem[b])
        if j + 2 < 8:
            gh[b] = start_gather(j + 2)
    pending_s[0].wait()
    pending_s[1].wait()


def _b2(eidx, counts, x, y):
    f = pl.kernel(
        _b2_body,
        out_type=[
            jax.ShapeDtypeStruct((CAP, DIM), jnp.float32),
            jax.ShapeDtypeStruct((CAP, DIM), jnp.float32),
            jax.ShapeDtypeStruct((NPAIR,), jnp.int32),
            jax.ShapeDtypeStruct((48,), jnp.int32),
        ],
        mesh=_MESH,
        scratch_types=[
            pltpu.VMEM((PP,), jnp.int32),
            pltpu.VMEM((NW, 1, EPAD), jnp.int32),
            pltpu.VMEM((PP,), jnp.int32),
            pltpu.VMEM((48,), jnp.int32),
            pltpu.VMEM((4, 32), jnp.int32),
            pltpu.VMEM((4, 32), jnp.int32),
            pltpu.VMEM((32, DIM), jnp.float32),
            pltpu.VMEM((32, DIM), jnp.float32),
            pltpu.SemaphoreType.DMA,
            pltpu.SemaphoreType.DMA,
            pltpu.SemaphoreType.DMA,
            pltpu.SemaphoreType.DMA,
        ],
    )
    return f(eidx, counts, x, y)


def _c_body(texp_ref, ys_ref, xs_ref, w_ref, out_ref):
    w3 = w_ref[0]
    out_ref[:, :DIM] = jnp.dot(ys_ref[...], w3[:, :DIM],
                               preferred_element_type=jnp.float32)
    out_ref[:, DIM:] = jnp.dot(xs_ref[...], w3[:, DIM:],
                               preferred_element_type=jnp.float32)


def _c(texp, ys_s, xs_s, W_qkv):
    grid_spec = pltpu.PrefetchScalarGridSpec(
        num_scalar_prefetch=1,
        grid=(NT,),
        in_specs=[
            pl.BlockSpec((TM, DIM), lambda r, texp: (r, 0)),
            pl.BlockSpec((TM, DIM), lambda r, texp: (r, 0)),
            pl.BlockSpec((1, DIM, 3 * DIM), lambda r, texp: (texp[r], 0, 0)),
        ],
        out_specs=pl.BlockSpec((TM, 3 * DIM), lambda r, texp: (r, 0)),
    )
    return pl.pallas_call(
        _c_body,
        grid_spec=grid_spec,
        out_shape=jax.ShapeDtypeStruct((CAP, 3 * DIM), jnp.float32),
        compiler_params=pltpu.CompilerParams(
            dimension_semantics=("arbitrary",)),
    )(texp, ys_s, xs_s, W_qkv)


def _d_body(pairout_hbm, pos_hbm, tv_hbm, qr_hbm, kr_hbm, vr_hbm,
            pos_buf, tv_buf, rows0, rows1, obq, obk, obv, sem0, sem1):
    w = _wid()
    pltpu.sync_copy(pos_hbm.at[pl.ds(w * PP, PP)], pos_buf)
    pltpu.sync_copy(tv_hbm.at[pl.ds(w * PP, PP)], tv_buf)
    obs = [obq, obk, obv]
    outs = [qr_hbm, kr_hbm, vr_hbm]
    bufs = [rows0, rows1]
    sems = [sem0, sem1]

    def start(c):
        posv = pos_buf[pl.ds(c * 16, 16)]
        return pltpu.async_copy(pairout_hbm.at[posv], bufs[c % 2],
                                sems[c % 2])

    nch = TOKW // 8
    gh = [start(0), start(1)]
    for c in range(nch):
        rows = bufs[c % 2]
        gh[c % 2].wait()
        gv = tv_buf[pl.ds(c * 16, 16)]
        g = [gv[j] for j in range(16)]

        for pi in range(3):
            def body(s, _, rows=rows, g=g, pi=pi):
                for t in range(8):
                    a = rows[2 * t, pl.ds(pi * DIM + s * 16, 16)]
                    b = rows[2 * t + 1, pl.ds(pi * DIM + s * 16, 16)]
                    obs[pi][t, pl.ds(s * 16, 16)] = (
                        g[2 * t] * a + g[2 * t + 1] * b)
                return 0

            lax.fori_loop(0, DIM // 16, body, 0)
        rb = w * TOKW + c * 8
        for pi in range(3):
            pltpu.sync_copy(obs[pi], outs[pi].at[pl.ds(rb, 8)])
        if c + 2 < nch:
            gh[c % 2] = start(c + 2)


def _d(pairout, pos, tvals):
    f = pl.kernel(
        _d_body,
        out_type=[
            jax.ShapeDtypeStruct((B, DIM), jnp.float32),
            jax.ShapeDtypeStruct((B, DIM), jnp.float32),
            jax.ShapeDtypeStruct((B, DIM), jnp.float32),
        ],
        mesh=_MESH,
        scratch_types=[
            pltpu.VMEM((PP,), jnp.int32),
            pltpu.VMEM((PP,), jnp.float32),
            pltpu.VMEM((16, 3 * DIM), jnp.float32),
            pltpu.VMEM((16, 3 * DIM), jnp.float32),
            pltpu.VMEM((8, DIM), jnp.float32),
            pltpu.VMEM((8, DIM), jnp.float32),
            pltpu.VMEM((8, DIM), jnp.float32),
            pltpu.SemaphoreType.DMA,
            pltpu.SemaphoreType.DMA,
        ],
    )
    return f(pairout, pos, tvals)


def _attn_body(q_ref, k_ref, v_ref, o_ref):
    scale = DH ** -0.5
    r = jax.lax.broadcasted_iota(jnp.int32, (1, 128, 128), 1) // H
    c = jax.lax.broadcasted_iota(jnp.int32, (1, 128, 128), 2) // H
    amask = jnp.where(r == c, 0.0, NEG).astype(jnp.float32)
    nb = q_ref.shape[0] // 128
    q3 = q_ref[...].reshape(nb, 128, DH)
    k3 = k_ref[...].reshape(nb, 128, DH)
    v3 = v_ref[...].reshape(nb, 128, DH)
    s = jax.lax.dot_general(q3, k3, (((2,), (2,)), ((0,), (0,))),
                            preferred_element_type=jnp.float32)
    s = s * scale + amask
    m = jnp.max(s, axis=-1, keepdims=True)
    p = jnp.exp(s - m)
    attn = p / jnp.sum(p, axis=-1, keepdims=True)
    o = jax.lax.dot_general(attn, v3, (((2,), (1,)), ((0,), (0,))),
                            preferred_element_type=jnp.float32)
    o_ref[...] = o.reshape(nb * 128, DH)


def _proj_body(o_ref, wp_ref, bp_ref, out_ref):
    out_ref[...] = jnp.dot(o_ref[...], wp_ref[...],
                           preferred_element_type=jnp.float32) + bp_ref[0]


def _attention_and_proj(qp, kp, vp, Wp, bp):
    q_r = qp.reshape(B * H, DH)
    k_r = kp.reshape(B * H, DH)
    v_r = vp.reshape(B * H, DH)
    RB = 8192
    o_r = pl.pallas_call(
        _attn_body,
        grid=(B * H // RB,),
        in_specs=[
            pl.BlockSpec((RB, DH), lambda i: (i, 0)),
            pl.BlockSpec((RB, DH), lambda i: (i, 0)),
            pl.BlockSpec((RB, DH), lambda i: (i, 0)),
        ],
        out_specs=pl.BlockSpec((RB, DH), lambda i: (i, 0)),
        out_shape=jax.ShapeDtypeStruct((B * H, DH), jnp.float32),
    )(q_r, k_r, v_r)

    o_flat = o_r.reshape(B, DIM)
    wp_perm = Wp.T.reshape(DH, H, DIM).transpose(1, 0, 2).reshape(DIM, DIM)
    return pl.pallas_call(
        _proj_body,
        grid=(B // TBLK,),
        in_specs=[
            pl.BlockSpec((TBLK, DIM), lambda i: (i, 0)),
            pl.BlockSpec((DIM, DIM), lambda i: (0, 0)),
            pl.BlockSpec((1, DIM), lambda i: (0, 0)),
        ],
        out_specs=pl.BlockSpec((TBLK, DIM), lambda i: (i, 0)),
        out_shape=jax.ShapeDtypeStruct((B, DIM), jnp.float32),
    )(o_flat, wp_perm, bp[None, :])


def kernel(x, y, W_qkv, Wg, bg, Wp, bp, expert_bias):
    eidx, tvals, counts = _gating(x, Wg, bg, expert_bias)
    xs_s, ys_s, pos, texp = _b2(eidx, counts, x, y)
    pairout = _c(texp, ys_s, xs_s, W_qkv)
    q_r, k_r, v_r = _d(pairout, pos, tvals)
    return _attention_and_proj(q_r, k_r, v_r, Wp, bp)
